# baseline (speedup 1.0000x reference)
import functools

import jax
import jax.numpy as jnp
import numpy as np
from jax import lax
from jax.experimental import pallas as pl
from jax.experimental.pallas import tpu as pltpu
from jax.experimental.pallas import tpu_sc as plsc

NB = 8
N = 2048
NPAIR = 3
NTASK = 24
QI = 512
JBV = 4
IBU = 2

_INF = float(np.inf)


def _rot16(x, shift):
    idx = lax.bitwise_and(lax.iota(jnp.int32, 16) + shift, 15)
    dnums = lax.GatherDimensionNumbers(
        offset_dims=(), collapsed_slice_dims=(0,), start_index_map=(0,))
    return lax.gather(x, idx[:, None], dnums, (1,),
                      mode=lax.GatherScatterMode.PROMISE_IN_BOUNDS)


def _lane_min_all(m):
    for shift in (8, 4, 2, 1):
        m = jnp.minimum(m, _rot16(m, shift))
    return m


def _sqrt16(x):
    xc = jnp.maximum(x, jnp.float32(1e-30))
    i = lax.bitcast_convert_type(xc, jnp.int32)
    i = jnp.int32(0x5F3759DF) - lax.shift_right_logical(i, 1)
    z = lax.bitcast_convert_type(i, jnp.float32)
    for _ in range(3):
        z = z * (jnp.float32(1.5) - jnp.float32(0.5) * xc * z * z)
    return xc * z


def _chamfer_sc(a):
    mesh = plsc.VectorSubcoreMesh(core_axis_name="c", subcore_axis_name="s")

    @functools.partial(
        pl.kernel,
        out_type=jax.ShapeDtypeStruct((2, 16, 16), jnp.float32),
        mesh=mesh,
        scratch_types=[
            pltpu.VMEM((QI,), jnp.float32),
            pltpu.VMEM((QI,), jnp.float32),
            pltpu.VMEM((N,), jnp.float32),
            pltpu.VMEM((N,), jnp.float32),
            pltpu.VMEM((N,), jnp.float32),
            pltpu.VMEM((4, N), jnp.float32),
            pltpu.VMEM((4, 16), jnp.float32),
            pltpu.VMEM((16,), jnp.float32),
            pltpu.VMEM_SHARED((48, N), jnp.float32),
            pltpu.VMEM_SHARED((48, 16), jnp.float32),
        ],
    )
    def k(a_hbm, out_hbm, x0b, x1b, y0b, y1b, cmb, mgb, rsb, ob, spc, spr):
        c = lax.axis_index("c")
        s = lax.axis_index("s")

        for kk in range(3):
            q = s * 3 + kk
            t = q // 4
            qt = q % 4
            gt = c * 12 + t
            p = gt // NB
            b = gt % NB
            px = p // 2
            py = (p + 3) // 2

            pltpu.sync_copy(a_hbm.at[px, b, 0, pl.ds(qt * QI, QI)], x0b)
            pltpu.sync_copy(a_hbm.at[px, b, 1, pl.ds(qt * QI, QI)], x1b)
            pltpu.sync_copy(a_hbm.at[py, b, 0], y0b)
            pltpu.sync_copy(a_hbm.at[py, b, 1], y1b)

            def init_body(jv, _):
                cmb[pl.ds(jv * 16, 16)] = jnp.full((16,), _INF, jnp.float32)
                return 0
            lax.fori_loop(0, N // 16, init_body, 0)

            def ig_body(ig, sumv):
                xv0 = x0b[pl.ds(ig * 16, 16)]
                xv1 = x1b[pl.ds(ig * 16, 16)]
                rmv = jnp.full((16,), _INF, jnp.float32)
                for u in range(0, 16, IBU):
                    xs = [(xv0[u + w], xv1[u + w]) for w in range(IBU)]

                    def j_body(jb, accs):
                        accs = [list(av) for av in accs]
                        for v in range(JBV):
                            jo = jb * (JBV * 16) + v * 16
                            yv0 = y0b[pl.ds(jo, 16)]
                            yv1 = y1b[pl.ds(jo, 16)]
                            cmv = cmb[pl.ds(jo, 16)]
                            for w in range(IBU):
                                d0 = yv0 - xs[w][0]
                                d1 = yv1 - xs[w][1]
                                dsq = d0 * d0 + d1 * d1
                                accs[w][v] = jnp.minimum(accs[w][v], dsq)
                                cmv = jnp.minimum(cmv, dsq)
                            cmb[pl.ds(jo, 16)] = cmv
                        return tuple(tuple(av) for av in accs)

                    inf16 = jnp.full((16,), _INF, jnp.float32)
                    accs = lax.fori_loop(0, N // (JBV * 16), j_body,
                                         ((inf16,) * JBV,) * IBU)
                    for w in range(IBU):
                        m = jnp.minimum(jnp.minimum(accs[w][0], accs[w][1]),
                                        jnp.minimum(accs[w][2], accs[w][3]))
                        ms = _lane_min_all(m)
                        lane = lax.iota(jnp.int32, 16) == (u + w)
                        rmv = jnp.where(lane, ms, rmv)
                return sumv + _sqrt16(rmv)

            sumv = lax.fori_loop(0, QI // 16, ig_body,
                                 jnp.zeros((16,), jnp.float32))
            ob[...] = sumv
            pltpu.sync_copy(cmb, spc.at[q])
            pltpu.sync_copy(ob, spr.at[q])

        plsc.subcore_barrier()

        @pl.when(s < 12)
        def _merge():
            t = s
            gt = c * 12 + t
            pltpu.sync_copy(spc.at[pl.ds(t * 4, 4)], mgb)
            pltpu.sync_copy(spr.at[pl.ds(t * 4, 4)], rsb)

            def m_body(jv, acc):
                jo = jv * 16
                m = jnp.minimum(jnp.minimum(mgb[0, pl.ds(jo, 16)],
                                            mgb[1, pl.ds(jo, 16)]),
                                jnp.minimum(mgb[2, pl.ds(jo, 16)],
                                            mgb[3, pl.ds(jo, 16)]))
                return acc + _sqrt16(m)

            csum = lax.fori_loop(0, N // 16, m_body,
                                 jnp.zeros((16,), jnp.float32))
            total = csum + rsb[0] + rsb[1] + rsb[2] + rsb[3]
            ob[...] = total

        @pl.when(s >= 12)
        def _zero():
            ob[...] = jnp.zeros((16,), jnp.float32)

        pltpu.sync_copy(ob, out_hbm.at[c, s])

    return k(a)


def kernel(point_set1, point_set2, point_set3):
    a = jnp.stack([point_set1, point_set2, point_set3])
    a = jnp.transpose(a, (0, 1, 3, 2))
    sums = _chamfer_sc(a)
    task_tot = sums[:, :12, :].sum(-1).reshape(NTASK)
    dist = task_tot.reshape(NPAIR, NB) / (2.0 * N)
    return jnp.mean(1.0 - dist, axis=0)


# speedup vs baseline: 1.2779x; 1.2779x over previous
import functools

import jax
import jax.numpy as jnp
import numpy as np
from jax import lax
from jax.experimental import pallas as pl
from jax.experimental.pallas import tpu as pltpu
from jax.experimental.pallas import tpu_sc as plsc

NB = 8
N = 2048
NPAIR = 3
NTASK = 24
NCHUNK = 96
QI = 512
JBV = 8
IBU = 2
TILES = N // 256
ROW_TILES = QI * 32 // 256

_INF = float(np.inf)


def _sqrt16(x):
    xc = jnp.maximum(x, jnp.float32(1e-30))
    i = lax.bitcast_convert_type(xc, jnp.int32)
    i = jnp.int32(0x5F3759DF) - lax.shift_right_logical(i, 1)
    z = lax.bitcast_convert_type(i, jnp.float32)
    for _ in range(3):
        z = z * (jnp.float32(1.5) - jnp.float32(0.5) * xc * z * z)
    return xc * z


def _mesh():
    return plsc.VectorSubcoreMesh(core_axis_name="c", subcore_axis_name="s")


def _stage_a(xsp, abf):

    @functools.partial(
        pl.kernel,
        out_type=jax.ShapeDtypeStruct((NCHUNK, ROW_TILES + TILES, 2, 128),
                                      jnp.bfloat16),
        mesh=_mesh(),
        scratch_types=[
            pltpu.VMEM((QI // 8, 2, 128), jnp.bfloat16),
            pltpu.VMEM((QI // 8, 2, 128), jnp.bfloat16),
            pltpu.VMEM((TILES, 2, 128), jnp.bfloat16),
            pltpu.VMEM((TILES, 2, 128), jnp.bfloat16),
            pltpu.VMEM((TILES, 2, 128), jnp.bfloat16),
            pltpu.VMEM((ROW_TILES, 2, 128), jnp.bfloat16),
        ],
    )
    def k(xsp_hbm, abf_hbm, out_hbm, x0b, x1b, y0b, y1b, cmb, rwb):
        wid = lax.axis_index("c") * 16 + lax.axis_index("s")
        for kk in range(3):
            ch = wid * 3 + kk
            t = ch // 4
            qt = ch % 4
            p = t // NB
            b = t % NB
            px = p // 2
            py = (p + 3) // 2

            xrow = (px * NB + b) * 2
            yrow = (py * NB + b) * 2
            pltpu.sync_copy(
                xsp_hbm.at[xrow, pl.ds(qt * (QI // 8), QI // 8)], x0b)
            pltpu.sync_copy(
                xsp_hbm.at[xrow + 1, pl.ds(qt * (QI // 8), QI // 8)], x1b)
            pltpu.sync_copy(abf_hbm.at[yrow], y0b)
            pltpu.sync_copy(abf_hbm.at[yrow + 1], y1b)

            inf216 = jnp.full((2, 16), _INF, jnp.bfloat16)

            def init_body(jt, _):
                for h in range(8):
                    cmb[jt, :, pl.ds(h * 16, 16)] = inf216
                return 0
            lax.fori_loop(0, TILES, init_body, 0)

            def ig_body(ig, _):
                for u in range(0, 16, IBU):
                    xs = []
                    for w in range(IBU):
                        i = ig * 16 + u + w
                        xsl = (i // 8, slice(None), pl.ds((i % 8) * 16, 16))
                        xs.append((x0b[xsl], x1b[xsl]))

                    def j_body(jt, accs):
                        accs = [list(av) for av in accs]
                        for h in range(8):
                            sl = (jt, slice(None), pl.ds(h * 16, 16))
                            yv0 = y0b[sl]
                            yv1 = y1b[sl]
                            cmv = cmb[sl]
                            for w in range(IBU):
                                d0 = yv0 - xs[w][0]
                                d1 = yv1 - xs[w][1]
                                dsq = d0 * d0 + d1 * d1
                                accs[w][h] = jnp.minimum(accs[w][h], dsq)
                                cmv = jnp.minimum(cmv, dsq)
                            cmb[sl] = cmv
                        return tuple(tuple(av) for av in accs)

                    accs = lax.fori_loop(0, TILES, j_body,
                                         ((inf216,) * 8,) * IBU)
                    for w in range(IBU):
                        m = accs[w][0]
                        for h in range(1, 8):
                            m = jnp.minimum(m, accs[w][h])
                        i = ig * 16 + u + w
                        rwb[i // 8, :, pl.ds((i % 8) * 16, 16)] = m
                return 0

            lax.fori_loop(0, QI // 16, ig_body, 0)
            pltpu.sync_copy(rwb, out_hbm.at[ch, pl.ds(0, ROW_TILES)])
            pltpu.sync_copy(cmb, out_hbm.at[ch, pl.ds(ROW_TILES, TILES)])

    return k(xsp, abf)


def _stage_b(rowt, colq):

    @functools.partial(
        pl.kernel,
        out_type=jax.ShapeDtypeStruct((128, 16), jnp.float32),
        mesh=_mesh(),
        scratch_types=[
            pltpu.VMEM((32, QI), jnp.float32),
            pltpu.VMEM((4, N), jnp.float32),
            pltpu.VMEM((16,), jnp.float32),
        ],
    )
    def k(rowt_hbm, colq_hbm, out_hbm, rb, cb, ob):
        c = lax.axis_index("c")
        s = lax.axis_index("s")
        wid = c * 16 + s

        for kk in range(3):
            ch = wid * 3 + kk
            pltpu.sync_copy(rowt_hbm.at[ch], rb)

            def g_body(g, acc):
                m = rb[0, pl.ds(g * 16, 16)]
                for kq in range(1, 32):
                    m = jnp.minimum(m, rb[kq, pl.ds(g * 16, 16)])
                return acc + _sqrt16(m)

            acc = lax.fori_loop(0, QI // 16, g_body,
                                jnp.zeros((16,), jnp.float32))
            ob[...] = acc
            pltpu.sync_copy(ob, out_hbm.at[ch])

        @pl.when(s < 12)
        def _col():
            t = c * 12 + s
            pltpu.sync_copy(colq_hbm.at[t], cb)

            def m_body(jv, acc):
                jo = jv * 16
                m = jnp.minimum(jnp.minimum(cb[0, pl.ds(jo, 16)],
                                            cb[1, pl.ds(jo, 16)]),
                                jnp.minimum(cb[2, pl.ds(jo, 16)],
                                            cb[3, pl.ds(jo, 16)]))
                return acc + _sqrt16(m)

            csum = lax.fori_loop(0, N // 16, m_body,
                                 jnp.zeros((16,), jnp.float32))
            ob[...] = csum
            pltpu.sync_copy(ob, out_hbm.at[96 + t])

    return k(rowt, colq)


def kernel(point_set1, point_set2, point_set3):
    a = jnp.stack([point_set1, point_set2, point_set3])
    a = jnp.transpose(a, (0, 1, 3, 2))
    xbf = a.astype(jnp.bfloat16).reshape(3 * NB * 2, N)
    abf = xbf.reshape(3 * NB * 2, TILES, 2, 128)
    xsp = jnp.broadcast_to(xbf.reshape(3 * NB * 2, N // 8, 1, 8, 1),
                           (3 * NB * 2, N // 8, 2, 8, 16))
    xsp = xsp.reshape(3 * NB * 2, N // 8, 2, 128)

    raw = _stage_a(xsp, abf).astype(jnp.float32)
    rows = raw[:, :ROW_TILES].reshape(NCHUNK, ROW_TILES, 2, 8, 16)
    rowt = jnp.transpose(rows, (0, 2, 4, 1, 3)).reshape(NCHUNK, 32, QI)
    colq = raw[:, ROW_TILES:].reshape(NCHUNK, N).reshape(NTASK, 4, N)

    sums = _stage_b(rowt, colq)
    rowsum = sums[:NCHUNK].sum(-1).reshape(NTASK, 4).sum(-1)
    colsum = sums[NCHUNK:NCHUNK + NTASK].sum(-1)
    dist = (rowsum + colsum).reshape(NPAIR, NB) / (2.0 * N)
    return jnp.mean(1.0 - dist, axis=0)
